# baseline (speedup 1.0000x reference)
import functools

import jax
import jax.numpy as jnp
from jax import lax
from jax.experimental import pallas as pl
from jax.experimental.pallas import tpu as pltpu
from jax.experimental.pallas import tpu_sc as plsc

_NC, _NS = 2, 16
_NW = _NC * _NS
_H = 128
_CW = 16
_IC = 4


def _make_sc_agg(n_pad, rpw, rows_w, mode):
    gather = mode == "acc"
    mesh = plsc.VectorSubcoreMesh(core_axis_name="c", subcore_axis_name="s")

    out_type = jax.ShapeDtypeStruct((_NC, _NS, rpw, _H), jnp.float32)
    scratch = [
        pltpu.VMEM((_IC, 128), jnp.int32),
        pltpu.VMEM((_IC, 128), jnp.int32),
        pltpu.VMEM((128, _H), jnp.float32),
        pltpu.VMEM_SHARED((n_pad, _H), jnp.float32),
        pltpu.SemaphoreType.DMA,
    ]
    if gather:
        scratch += [
            pltpu.VMEM((2, 64), jnp.int32),
            pltpu.VMEM((2, 64), jnp.int32),
            pltpu.SemaphoreType.DMA,
        ]

    assert rpw % 128 == 0
    n_chunks = rpw // 128

    def body(*refs):
        if gather:
            (src_hbm, dst_hbm, tab_hbm, acc_out,
             srcv, dstv, rowsv, acc_sh, sem, src64, dst64, sem2) = refs
        else:
            (dst_hbm, acc_out, srcv, dstv, rowsv, acc_sh, sem) = refs
        cid = lax.axis_index("c")
        sid = lax.axis_index("s")
        wid = cid * _NS + sid

        iota16 = lax.iota(jnp.int32, 16)

        def set_idx_row(base):
            for j in range(_H // 16):
                srcv[0, pl.ds(j * 16, 16)] = base + j * 16 + iota16

        def fill(val):
            v16 = jnp.full((16,), val, jnp.float32)

            def frow(i, carry):
                for j in range(_H // 16):
                    rowsv[i, pl.ds(j * 16, 16)] = v16
                return carry

            lax.fori_loop(0, 128, frow, 0)

        fill(0.0)

        def zinit(k, carry):
            set_idx_row(sid * rpw + k * 128)
            pltpu.sync_copy(rowsv, acc_sh.at[srcv.at[0]])
            return carry

        lax.fori_loop(0, n_chunks, zinit, 0)
        if not gather:
            fill(1.0)

        plsc.subcore_barrier()

        if gather:
            sems = (sem, sem2)
            nh = 2 * _IC

            def prep(hh):
                r, half = hh // 2, hh % 2
                for j in range(4):
                    src64[hh % 2, pl.ds(j * 16, 16)] = (
                        srcv[r, pl.ds(half * 64 + j * 16, 16)])
                    dst64[hh % 2, pl.ds(j * 16, 16)] = (
                        dstv[r, pl.ds(half * 64 + j * 16, 16)])

            def start(hh):
                return pltpu.async_copy(
                    tab_hbm.at[src64.at[hh % 2]],
                    rowsv.at[pl.ds((hh % 2) * 64, 64)], sems[hh % 2])

            def chunk(ci, carry):
                r0 = pl.multiple_of(ci * _IC, _IC)
                pltpu.sync_copy(src_hbm.at[wid, pl.ds(r0, _IC)], srcv)
                pltpu.sync_copy(dst_hbm.at[wid, pl.ds(r0, _IC)], dstv)
                prep(0)
                d = start(0)
                for hh in range(nh):
                    d.wait()
                    if hh + 1 < nh:
                        prep(hh + 1)
                        d = start(hh + 1)
                    pltpu.sync_copy(rowsv.at[pl.ds((hh % 2) * 64, 64)],
                                    acc_sh.at[dst64.at[hh % 2]], add=True)
                return carry

            lax.fori_loop(0, rows_w // _IC, chunk, 0)
        else:
            def chunk(ci, carry):
                r0 = pl.multiple_of(ci * _IC, _IC)
                pltpu.sync_copy(dst_hbm.at[wid, pl.ds(r0, _IC)], dstv)

                def step(i, c2):
                    pltpu.sync_copy(rowsv, acc_sh.at[dstv.at[i]], add=True)
                    return c2

                return lax.fori_loop(0, _IC, step, carry)

            lax.fori_loop(0, rows_w // _IC, chunk, 0)

        plsc.subcore_barrier()

        def wb(k, carry):
            off = pl.multiple_of(k * 128, 128)
            set_idx_row(sid * rpw + off)
            pltpu.async_copy(acc_sh.at[srcv.at[0]], rowsv, sem).wait()
            pltpu.sync_copy(rowsv, acc_out.at[cid, sid, pl.ds(off, 128)])
            return carry

        lax.fori_loop(0, n_chunks, wb, 0)

    return pl.kernel(body, out_type, mesh=mesh, scratch_types=scratch)


def _proj_body(x_ref, w_ref, b_ref, o_ref):
    o_ref[...] = (jnp.dot(x_ref[...], w_ref[...],
                          preferred_element_type=jnp.float32) + b_ref[...])


def _tc_proj(x, w, b, bm):
    m, k = x.shape
    n = w.shape[1]
    return pl.pallas_call(
        _proj_body,
        grid=(m // bm,),
        in_specs=[
            pl.BlockSpec((bm, k), lambda i: (i, 0)),
            pl.BlockSpec((k, n), lambda i: (0, 0)),
            pl.BlockSpec((1, n), lambda i: (0, 0)),
        ],
        out_specs=pl.BlockSpec((bm, n), lambda i: (i, 0)),
        out_shape=jax.ShapeDtypeStruct((m, n), jnp.float32),
    )(x, w, b.reshape(1, n))


def _layer_body(p0_ref, p1_ref, c0_ref, c1_ref, x_ref, wl_ref, bl_ref, wr_ref,
                o_ref):
    cnt = jnp.maximum(c0_ref[:, 0:1] + c1_ref[:, 0:1], 1.0)
    mean = (p0_ref[...] + p1_ref[...]) / cnt
    h = (jnp.dot(mean, wl_ref[...], preferred_element_type=jnp.float32)
         + bl_ref[...]
         + jnp.dot(x_ref[...], wr_ref[...], preferred_element_type=jnp.float32))
    o_ref[...] = jnp.maximum(h, 0.0)


def _tc_layer(p0, p1, c0, c1, x, wl, bl, wr, bm):
    m = x.shape[0]
    h = wl.shape[1]
    return pl.pallas_call(
        _layer_body,
        grid=(m // bm,),
        in_specs=[
            pl.BlockSpec((bm, _H), lambda i: (i, 0)),
            pl.BlockSpec((bm, _H), lambda i: (i, 0)),
            pl.BlockSpec((bm, _H), lambda i: (i, 0)),
            pl.BlockSpec((bm, _H), lambda i: (i, 0)),
            pl.BlockSpec((bm, _H), lambda i: (i, 0)),
            pl.BlockSpec((_H, h), lambda i: (0, 0)),
            pl.BlockSpec((1, h), lambda i: (0, 0)),
            pl.BlockSpec((_H, h), lambda i: (0, 0)),
        ],
        out_specs=pl.BlockSpec((bm, h), lambda i: (i, 0)),
        out_shape=jax.ShapeDtypeStruct((m, h), jnp.float32),
    )(p0, p1, c0, c1, x, wl, bl.reshape(1, h), wr)


def _head_body(p0_ref, p1_ref, c0_ref, c1_ref, e1_ref, xe_ref,
               wl_ref, bl_ref, wr_ref, wc1_ref, bc1_ref, wc2_ref, bc2_ref,
               wctl_ref, bctl_ref, wtrt_ref, btrt_ref, wT_ref, bT_ref,
               woc_ref, boc_ref, wot_ref, bot_ref, woT_ref, boT_ref,
               ot1_ref, ot0_ref, oT_ref, ht1_ref, ht0_ref):
    dot = functools.partial(jnp.dot, preferred_element_type=jnp.float32)
    cnt = jnp.maximum(c0_ref[:, 0:1] + c1_ref[:, 0:1], 1.0)
    mean = (p0_ref[...] + p1_ref[...]) / cnt
    e1 = e1_ref[...]
    h2 = jnp.maximum(dot(mean, wl_ref[...]) + bl_ref[...]
                     + dot(e1, wr_ref[...]), 0.0)
    wc1 = wc1_ref[...]
    hidden = jnp.maximum(
        dot(xe_ref[...], wc1[0:_H]) + dot(e1, wc1[_H:2 * _H])
        + dot(h2, wc1[2 * _H:3 * _H]) + bc1_ref[...], 0.0)
    hidden = jnp.maximum(dot(hidden, wc2_ref[...]) + bc2_ref[...], 0.0)
    ht0 = jnp.maximum(dot(hidden, wctl_ref[...]) + bctl_ref[...], 0.0)
    ht1 = jnp.maximum(dot(hidden, wtrt_ref[...]) + btrt_ref[...], 0.0)
    hT = jnp.maximum(dot(hidden, wT_ref[...]) + bT_ref[...], 0.0)
    ot0_ref[...] = jnp.maximum(dot(ht0, woc_ref[...]) + boc_ref[...], 0.0)
    ot1_ref[...] = jnp.maximum(dot(ht1, wot_ref[...]) + bot_ref[...], 0.0)
    oT_ref[...] = jnp.maximum(dot(hT, woT_ref[...]) + boT_ref[...], 0.0)
    ht1_ref[...] = ht1
    ht0_ref[...] = ht0


def _tc_head(p0, p1, c0, c1, e1, xe, wl, bl, wr, wc1, bc1, wc2, bc2,
             wctl, bctl, wtrt, btrt, wT, bT, woc, boc, wot, bot, woT, boT,
             nu, bm):
    hh = wctl.shape[1]
    out = wot.shape[1]

    def rows(i):
        return (i, 0)

    def whole(i):
        return (0, 0)

    in_specs = [
        pl.BlockSpec((bm, _H), rows),
        pl.BlockSpec((bm, _H), rows),
        pl.BlockSpec((bm, _H), rows),
        pl.BlockSpec((bm, _H), rows),
        pl.BlockSpec((bm, _H), rows),
        pl.BlockSpec((bm, _H), rows),
        pl.BlockSpec((_H, _H), whole),
        pl.BlockSpec((1, _H), whole),
        pl.BlockSpec((_H, _H), whole),
        pl.BlockSpec((3 * _H, _H), whole),
        pl.BlockSpec((1, _H), whole),
        pl.BlockSpec((_H, _H), whole),
        pl.BlockSpec((1, _H), whole),
        pl.BlockSpec((_H, hh), whole),
        pl.BlockSpec((1, hh), whole),
        pl.BlockSpec((_H, hh), whole),
        pl.BlockSpec((1, hh), whole),
        pl.BlockSpec((_H, hh), whole),
        pl.BlockSpec((1, hh), whole),
        pl.BlockSpec((hh, out), whole),
        pl.BlockSpec((1, out), whole),
        pl.BlockSpec((hh, out), whole),
        pl.BlockSpec((1, out), whole),
        pl.BlockSpec((hh, out), whole),
        pl.BlockSpec((1, out), whole),
    ]
    out_specs = [
        pl.BlockSpec((bm, out), rows),
        pl.BlockSpec((bm, out), rows),
        pl.BlockSpec((bm, out), rows),
        pl.BlockSpec((bm, hh), rows),
        pl.BlockSpec((bm, hh), rows),
    ]
    out_shape = [
        jax.ShapeDtypeStruct((nu, out), jnp.float32),
        jax.ShapeDtypeStruct((nu, out), jnp.float32),
        jax.ShapeDtypeStruct((nu, out), jnp.float32),
        jax.ShapeDtypeStruct((nu, hh), jnp.float32),
        jax.ShapeDtypeStruct((nu, hh), jnp.float32),
    ]
    return pl.pallas_call(
        _head_body,
        grid=(nu // bm,),
        in_specs=in_specs,
        out_specs=out_specs,
        out_shape=out_shape,
    )(p0, p1, c0, c1, e1, xe, wl, bl.reshape(1, _H), wr, wc1,
      bc1.reshape(1, _H), wc2, bc2.reshape(1, _H), wctl, bctl.reshape(1, hh),
      wtrt, btrt.reshape(1, hh), wT, bT.reshape(1, hh), woc,
      boc.reshape(1, out), wot, bot.reshape(1, out), woT, boT.reshape(1, out))


def kernel(xu, xp, edge_index, Wu, bu, Wp, bp, Wl0, bl0, Wr0, Wl1, bl1, Wr1,
           Wc1, bc1, Wc2, bc2, Wctl, bctl, Wtrt, btrt, WT, bT, Woc, boc,
           Wot, bot, WoT, boT):
    nu = xu.shape[0]
    npd = xp.shape[0]
    n = nu + npd
    e = edge_index.shape[1]

    rpw = -(-n // (_NS * 128)) * 128
    n_pad = _NS * rpw
    n_rows = -(-e // (128 * _NW * _IC)) * _NW * _IC
    rows_w = n_rows // _NW
    e_pad = n_rows * 128 - e

    pad_idx = n + jnp.arange(e_pad, dtype=jnp.int32) % (n_pad - n)
    src3d = jnp.concatenate(
        [edge_index[0], pad_idx]).reshape(_NW, rows_w, 128)
    dst3d = jnp.concatenate(
        [edge_index[1], pad_idx]).reshape(_NW, rows_w, 128)

    tab_pad = jnp.zeros((n_pad - n, _H), jnp.float32)

    xu_e = _tc_proj(xu, Wu, bu, bm=1000)
    xp_e = _tc_proj(xp, Wp, bp, bm=1000)
    table0 = jnp.concatenate([xu_e, xp_e, tab_pad], axis=0)

    cnt0 = _make_sc_agg(n_pad, rpw, rows_w, "cnt")(dst3d)
    agg0 = _make_sc_agg(n_pad, rpw, rows_w, "acc")(src3d, dst3d, table0)
    p = agg0.reshape(_NC, n_pad, _H)
    c = cnt0.reshape(_NC, n_pad, _H)

    table1 = _tc_layer(p[0], p[1], c[0], c[1], table0, Wl0, bl0, Wr0, bm=640)

    agg1 = _make_sc_agg(n_pad, rpw, rows_w, "acc")(src3d, dst3d, table1)
    q = agg1.reshape(_NC, n_pad, _H)

    ot1, ot0, oT, ht1, ht0 = _tc_head(
        q[0], q[1], c[0], c[1], table1, xu_e, Wl1, bl1, Wr1, Wc1, bc1,
        Wc2, bc2, Wctl, bctl, Wtrt, btrt, WT, bT, Woc, boc, Wot, bot,
        WoT, boT, nu, bm=1000)
    return (ot1, ot0, oT, ht1, ht0)


# speedup vs baseline: 1.0279x; 1.0279x over previous
import functools

import jax
import jax.numpy as jnp
from jax import lax
from jax.experimental import pallas as pl
from jax.experimental.pallas import tpu as pltpu
from jax.experimental.pallas import tpu_sc as plsc

_NC, _NS = 2, 16
_NW = _NC * _NS
_H = 128
_IC = 4


def _make_sc_agg(n_pad, rpw, rows_w, mode):
    gather = mode == "acc"
    mesh = plsc.VectorSubcoreMesh(core_axis_name="c", subcore_axis_name="s")

    out_type = jax.ShapeDtypeStruct((_NC, _NS, rpw, _H), jnp.float32)
    scratch = [
        pltpu.VMEM((_IC, 128), jnp.int32),
        pltpu.VMEM((_IC, 128), jnp.int32),
        pltpu.VMEM((128, _H), jnp.float32),
        pltpu.VMEM_SHARED((n_pad, _H), jnp.float32),
        pltpu.SemaphoreType.DMA,
    ]

    assert rpw % 128 == 0
    n_chunks = rpw // 128

    def body(*refs):
        if gather:
            (src_hbm, dst_hbm, tab_hbm, acc_out,
             srcv, dstv, rowsv, acc_sh, sem) = refs
        else:
            (dst_hbm, acc_out, srcv, dstv, rowsv, acc_sh, sem) = refs
        cid = lax.axis_index("c")
        sid = lax.axis_index("s")
        wid = cid * _NS + sid

        iota16 = lax.iota(jnp.int32, 16)

        def set_idx_row(base):
            for j in range(_H // 16):
                srcv[0, pl.ds(j * 16, 16)] = base + j * 16 + iota16

        def fill(val):
            v16 = jnp.full((16,), val, jnp.float32)

            def frow(i, carry):
                for j in range(_H // 16):
                    rowsv[i, pl.ds(j * 16, 16)] = v16
                return carry

            lax.fori_loop(0, 128, frow, 0)

        fill(0.0)

        def zinit(k, carry):
            set_idx_row(sid * rpw + k * 128)
            pltpu.sync_copy(rowsv, acc_sh.at[srcv.at[0]])
            return carry

        lax.fori_loop(0, n_chunks, zinit, 0)
        if not gather:
            fill(1.0)

        plsc.subcore_barrier()

        def chunk(ci, carry):
            r0 = pl.multiple_of(ci * _IC, _IC)
            if gather:
                pltpu.sync_copy(src_hbm.at[wid, pl.ds(r0, _IC)], srcv)
            pltpu.sync_copy(dst_hbm.at[wid, pl.ds(r0, _IC)], dstv)

            def step(i, c2):
                if gather:
                    pltpu.async_copy(tab_hbm.at[srcv.at[i]], rowsv, sem).wait()
                pltpu.sync_copy(rowsv, acc_sh.at[dstv.at[i]], add=True)
                return c2

            return lax.fori_loop(0, _IC, step, carry)

        lax.fori_loop(0, rows_w // _IC, chunk, 0)

        plsc.subcore_barrier()

        def wb(k, carry):
            off = pl.multiple_of(k * 128, 128)
            set_idx_row(sid * rpw + off)
            pltpu.async_copy(acc_sh.at[srcv.at[0]], rowsv, sem).wait()
            pltpu.sync_copy(rowsv, acc_out.at[cid, sid, pl.ds(off, 128)])
            return carry

        lax.fori_loop(0, n_chunks, wb, 0)

    return pl.kernel(body, out_type, mesh=mesh, scratch_types=scratch)


def _proj_body(x_ref, w_ref, b_ref, o_ref):
    o_ref[...] = (jnp.dot(x_ref[...], w_ref[...],
                          preferred_element_type=jnp.float32) + b_ref[...])


def _tc_proj(x, w, b, bm):
    m, k = x.shape
    n = w.shape[1]
    return pl.pallas_call(
        _proj_body,
        grid=(m // bm,),
        in_specs=[
            pl.BlockSpec((bm, k), lambda i: (i, 0)),
            pl.BlockSpec((k, n), lambda i: (0, 0)),
            pl.BlockSpec((1, n), lambda i: (0, 0)),
        ],
        out_specs=pl.BlockSpec((bm, n), lambda i: (i, 0)),
        out_shape=jax.ShapeDtypeStruct((m, n), jnp.float32),
    )(x, w, b.reshape(1, n))


def _layer_body(p0_ref, p1_ref, c0_ref, c1_ref, x_ref, wl_ref, bl_ref, wr_ref,
                o_ref):
    cnt = jnp.maximum(c0_ref[:, 0:1] + c1_ref[:, 0:1], 1.0)
    mean = (p0_ref[...] + p1_ref[...]) / cnt
    h = (jnp.dot(mean, wl_ref[...], preferred_element_type=jnp.float32)
         + bl_ref[...]
         + jnp.dot(x_ref[...], wr_ref[...], preferred_element_type=jnp.float32))
    o_ref[...] = jnp.maximum(h, 0.0)


def _tc_layer(p0, p1, c0, c1, x, wl, bl, wr, bm):
    m = x.shape[0]
    h = wl.shape[1]
    return pl.pallas_call(
        _layer_body,
        grid=(m // bm,),
        in_specs=[
            pl.BlockSpec((bm, _H), lambda i: (i, 0)),
            pl.BlockSpec((bm, _H), lambda i: (i, 0)),
            pl.BlockSpec((bm, _H), lambda i: (i, 0)),
            pl.BlockSpec((bm, _H), lambda i: (i, 0)),
            pl.BlockSpec((bm, _H), lambda i: (i, 0)),
            pl.BlockSpec((_H, h), lambda i: (0, 0)),
            pl.BlockSpec((1, h), lambda i: (0, 0)),
            pl.BlockSpec((_H, h), lambda i: (0, 0)),
        ],
        out_specs=pl.BlockSpec((bm, h), lambda i: (i, 0)),
        out_shape=jax.ShapeDtypeStruct((m, h), jnp.float32),
    )(p0, p1, c0, c1, x, wl, bl.reshape(1, h), wr)


def _head_body(p0_ref, p1_ref, c0_ref, c1_ref, e1_ref, xe_ref,
               wl_ref, bl_ref, wr_ref, wc1_ref, bc1_ref, wc2_ref, bc2_ref,
               wctl_ref, bctl_ref, wtrt_ref, btrt_ref, wT_ref, bT_ref,
               woc_ref, boc_ref, wot_ref, bot_ref, woT_ref, boT_ref,
               ot1_ref, ot0_ref, oT_ref, ht1_ref, ht0_ref):
    dot = functools.partial(jnp.dot, preferred_element_type=jnp.float32)
    cnt = jnp.maximum(c0_ref[:, 0:1] + c1_ref[:, 0:1], 1.0)
    mean = (p0_ref[...] + p1_ref[...]) / cnt
    e1 = e1_ref[...]
    h2 = jnp.maximum(dot(mean, wl_ref[...]) + bl_ref[...]
                     + dot(e1, wr_ref[...]), 0.0)
    wc1 = wc1_ref[...]
    hidden = jnp.maximum(
        dot(xe_ref[...], wc1[0:_H]) + dot(e1, wc1[_H:2 * _H])
        + dot(h2, wc1[2 * _H:3 * _H]) + bc1_ref[...], 0.0)
    hidden = jnp.maximum(dot(hidden, wc2_ref[...]) + bc2_ref[...], 0.0)
    ht0 = jnp.maximum(dot(hidden, wctl_ref[...]) + bctl_ref[...], 0.0)
    ht1 = jnp.maximum(dot(hidden, wtrt_ref[...]) + btrt_ref[...], 0.0)
    hT = jnp.maximum(dot(hidden, wT_ref[...]) + bT_ref[...], 0.0)
    ot0_ref[...] = jnp.maximum(dot(ht0, woc_ref[...]) + boc_ref[...], 0.0)
    ot1_ref[...] = jnp.maximum(dot(ht1, wot_ref[...]) + bot_ref[...], 0.0)
    oT_ref[...] = jnp.maximum(dot(hT, woT_ref[...]) + boT_ref[...], 0.0)
    ht1_ref[...] = ht1
    ht0_ref[...] = ht0


def _tc_head(p0, p1, c0, c1, e1, xe, wl, bl, wr, wc1, bc1, wc2, bc2,
             wctl, bctl, wtrt, btrt, wT, bT, woc, boc, wot, bot, woT, boT,
             nu, bm):
    hh = wctl.shape[1]
    out = wot.shape[1]

    def rows(i):
        return (i, 0)

    def whole(i):
        return (0, 0)

    in_specs = [
        pl.BlockSpec((bm, _H), rows),
        pl.BlockSpec((bm, _H), rows),
        pl.BlockSpec((bm, _H), rows),
        pl.BlockSpec((bm, _H), rows),
        pl.BlockSpec((bm, _H), rows),
        pl.BlockSpec((bm, _H), rows),
        pl.BlockSpec((_H, _H), whole),
        pl.BlockSpec((1, _H), whole),
        pl.BlockSpec((_H, _H), whole),
        pl.BlockSpec((3 * _H, _H), whole),
        pl.BlockSpec((1, _H), whole),
        pl.BlockSpec((_H, _H), whole),
        pl.BlockSpec((1, _H), whole),
        pl.BlockSpec((_H, hh), whole),
        pl.BlockSpec((1, hh), whole),
        pl.BlockSpec((_H, hh), whole),
        pl.BlockSpec((1, hh), whole),
        pl.BlockSpec((_H, hh), whole),
        pl.BlockSpec((1, hh), whole),
        pl.BlockSpec((hh, out), whole),
        pl.BlockSpec((1, out), whole),
        pl.BlockSpec((hh, out), whole),
        pl.BlockSpec((1, out), whole),
        pl.BlockSpec((hh, out), whole),
        pl.BlockSpec((1, out), whole),
    ]
    out_specs = [
        pl.BlockSpec((bm, out), rows),
        pl.BlockSpec((bm, out), rows),
        pl.BlockSpec((bm, out), rows),
        pl.BlockSpec((bm, hh), rows),
        pl.BlockSpec((bm, hh), rows),
    ]
    out_shape = [
        jax.ShapeDtypeStruct((nu, out), jnp.float32),
        jax.ShapeDtypeStruct((nu, out), jnp.float32),
        jax.ShapeDtypeStruct((nu, out), jnp.float32),
        jax.ShapeDtypeStruct((nu, hh), jnp.float32),
        jax.ShapeDtypeStruct((nu, hh), jnp.float32),
    ]
    return pl.pallas_call(
        _head_body,
        grid=(nu // bm,),
        in_specs=in_specs,
        out_specs=out_specs,
        out_shape=out_shape,
    )(p0, p1, c0, c1, e1, xe, wl, bl.reshape(1, _H), wr, wc1,
      bc1.reshape(1, _H), wc2, bc2.reshape(1, _H), wctl, bctl.reshape(1, hh),
      wtrt, btrt.reshape(1, hh), wT, bT.reshape(1, hh), woc,
      boc.reshape(1, out), wot, bot.reshape(1, out), woT, boT.reshape(1, out))


def kernel(xu, xp, edge_index, Wu, bu, Wp, bp, Wl0, bl0, Wr0, Wl1, bl1, Wr1,
           Wc1, bc1, Wc2, bc2, Wctl, bctl, Wtrt, btrt, WT, bT, Woc, boc,
           Wot, bot, WoT, boT):
    nu = xu.shape[0]
    npd = xp.shape[0]
    n = nu + npd
    e = edge_index.shape[1]

    rpw = -(-n // (_NS * 128)) * 128
    n_pad = _NS * rpw
    n_rows = -(-e // (128 * _NW * _IC)) * _NW * _IC
    rows_w = n_rows // _NW
    e_pad = n_rows * 128 - e

    pad_idx = n + jnp.arange(e_pad, dtype=jnp.int32) % (n_pad - n)
    src3d = jnp.concatenate(
        [edge_index[0], pad_idx]).reshape(_NW, rows_w, 128)
    dst3d = jnp.concatenate(
        [edge_index[1], pad_idx]).reshape(_NW, rows_w, 128)

    tab_pad = jnp.zeros((n_pad - n, _H), jnp.float32)

    xu_e = _tc_proj(xu, Wu, bu, bm=1000)
    xp_e = _tc_proj(xp, Wp, bp, bm=1000)
    table0 = jnp.concatenate([xu_e, xp_e, tab_pad], axis=0)

    cnt0 = _make_sc_agg(n_pad, rpw, rows_w, "cnt")(dst3d)
    agg0 = _make_sc_agg(n_pad, rpw, rows_w, "acc")(src3d, dst3d, table0)
    p = agg0.reshape(_NC, n_pad, _H)
    c = cnt0.reshape(_NC, n_pad, _H)

    table1 = _tc_layer(p[0], p[1], c[0], c[1], table0, Wl0, bl0, Wr0, bm=640)

    agg1 = _make_sc_agg(n_pad, rpw, rows_w, "acc")(src3d, dst3d, table1)
    q = agg1.reshape(_NC, n_pad, _H)

    ot1, ot0, oT, ht1, ht0 = _tc_head(
        q[0], q[1], c[0], c[1], table1, xu_e, Wl1, bl1, Wr1, Wc1, bc1,
        Wc2, bc2, Wctl, bctl, Wtrt, btrt, WT, bT, Woc, boc, Wot, bot,
        WoT, boT, nu, bm=1000)
    return (ot1, ot0, oT, ht1, ht0)
